# Initial kernel scaffold; baseline (speedup 1.0000x reference)
import jax
import jax.numpy as jnp
from jax.experimental import pallas as pl


def kernel(x, W, b_dec):
    raise NotImplementedError("write your pallas kernel here")


# speedup vs baseline: 20.0341x; 20.0341x over previous
import functools

import jax
import jax.numpy as jnp
from jax.experimental import pallas as pl
from jax.experimental.pallas import tpu as pltpu

ROWS_PER_BLOCK = 256
FEATURE_TILE = 2048
TOPK = 128
BISECT_ITERS = 22


def _body(x_ref, w_ref, o_ref, acts_ref, th_ref, *, nf):
    p = pl.program_id(1)
    j = pl.program_id(2)

    @pl.when(p == 0)
    def _encode():
        acts_ref[j] = jnp.dot(
            x_ref[...], w_ref[...], preferred_element_type=jnp.float32
        )

    @pl.when((p == 1) & (j == 0))
    def _threshold():
        rowmax = jnp.max(jnp.abs(acts_ref[0]), axis=1, keepdims=True)
        for n in range(1, nf):
            rowmax = jnp.maximum(
                rowmax, jnp.max(jnp.abs(acts_ref[n]), axis=1, keepdims=True)
            )

        def bisect_step(_, carry):
            lo, hi = carry
            mid = (lo + hi) * 0.5
            cnt = jnp.sum(jnp.abs(acts_ref[0]) >= mid, axis=1, keepdims=True)
            for n in range(1, nf):
                cnt = cnt + jnp.sum(
                    jnp.abs(acts_ref[n]) >= mid, axis=1, keepdims=True
                )
            pred = cnt >= TOPK
            lo = jnp.where(pred, mid, lo)
            hi = jnp.where(pred, hi, mid)
            return lo, hi

        lo0 = jnp.zeros_like(rowmax)
        hi0 = rowmax * 1.000001 + 1e-30
        lo, _ = jax.lax.fori_loop(0, BISECT_ITERS, bisect_step, (lo0, hi0))
        th_ref[...] = jnp.broadcast_to(lo, th_ref.shape)

    @pl.when(p == 1)
    def _decode():
        t = th_ref[:, 0:1]
        a = acts_ref[j]
        masked = jnp.where(jnp.abs(a) >= t, a, 0.0)
        contrib = jax.lax.dot_general(
            masked,
            w_ref[...],
            (((1,), (1,)), ((), ())),
            preferred_element_type=jnp.float32,
        )

        @pl.when(j == 0)
        def _():
            o_ref[...] = contrib

        @pl.when(j > 0)
        def _():
            o_ref[...] = o_ref[...] + contrib


def kernel(x, W, b_dec):
    batch, act_dim = x.shape
    _, dict_size = W.shape
    nr = batch // ROWS_PER_BLOCK
    nf = dict_size // FEATURE_TILE

    xb = x - b_dec[None, :]

    out = pl.pallas_call(
        functools.partial(_body, nf=nf),
        grid=(nr, 2, nf),
        in_specs=[
            pl.BlockSpec((ROWS_PER_BLOCK, act_dim), lambda i, p, j: (i, 0)),
            pl.BlockSpec((act_dim, FEATURE_TILE), lambda i, p, j: (0, j)),
        ],
        out_specs=pl.BlockSpec((ROWS_PER_BLOCK, act_dim), lambda i, p, j: (i, 0)),
        out_shape=jax.ShapeDtypeStruct((batch, act_dim), jnp.float32),
        scratch_shapes=[
            pltpu.VMEM((nf, ROWS_PER_BLOCK, FEATURE_TILE), jnp.float32),
            pltpu.VMEM((ROWS_PER_BLOCK, 128), jnp.float32),
        ],
        compiler_params=pltpu.CompilerParams(
            dimension_semantics=("arbitrary", "arbitrary", "arbitrary"),
        ),
    )(xb, W)

    return out + b_dec[None, :]
